# Initial kernel scaffold; baseline (speedup 1.0000x reference)
import jax
import jax.numpy as jnp
from jax.experimental import pallas as pl


def kernel(cls_logits, box_preds, lmk_preds, gaze_preds, default_boxes, gt_boxes, gt_landmarks, gt_gaze, gt_labels):
    raise NotImplementedError("write your pallas kernel here")


# speedup vs baseline: 60.5314x; 60.5314x over previous
import jax
import jax.numpy as jnp
from jax.experimental import pallas as pl
from jax.experimental.pallas import tpu as pltpu

_B, _A, _G, _C = 16, 24564, 8, 3
_AP = 24576
_S, _L = 192, 128
_NEG_POS_RATIO = 3
_IOU_THR = 0.5
_I32_MIN = -2147483648
_I32_MAXP = 2147483647


def _smooth_l1(x):
    a = jnp.abs(x)
    return jnp.where(a < 1.0, 0.5 * a * a, a - 0.5)


def _body(cls_ref, box_ref, lmk_ref, gaze_ref, db_ref, gtb, gtl, gtlmk, gtgz,
          out_ref, acc):
    b = pl.program_id(0)

    @pl.when(b == 0)
    def _init():
        for i in range(6):
            acc[i] = 0.0

    dcx = db_ref[0]
    dcy = db_ref[1]
    dw = db_ref[2]
    dh = db_ref[3]
    dx1 = dcx - dw / 2.0
    dy1 = dcy - dh / 2.0
    dx2 = dcx + dw / 2.0
    dy2 = dcy + dh / 2.0
    area_b = (dx2 - dx1) * (dy2 - dy1)

    sl = jax.lax.broadcasted_iota(jnp.int32, (_S, _L), 0)
    ll = jax.lax.broadcasted_iota(jnp.int32, (_S, _L), 1)
    aid = sl * _L + ll
    valid = aid < _A

    ious = []
    best_iou = None
    best_g = None
    for g in range(_G):
        gx1 = gtb[0, g, 0]
        gy1 = gtb[0, g, 1]
        gx2 = gtb[0, g, 2]
        gy2 = gtb[0, g, 3]
        w = jnp.maximum(jnp.minimum(gx2, dx2) - jnp.maximum(gx1, dx1), 0.0)
        h = jnp.maximum(jnp.minimum(gy2, dy2) - jnp.maximum(gy1, dy1), 0.0)
        inter = w * h
        area_a = (gx2 - gx1) * (gy2 - gy1)
        union = jnp.maximum(area_a + area_b - inter, 1e-8)
        iou = inter / union
        ious.append(iou)
        if g == 0:
            best_iou = iou
            best_g = jnp.zeros((_S, _L), jnp.int32)
        else:
            upd = iou > best_iou
            best_g = jnp.where(upd, g, best_g)
            best_iou = jnp.where(upd, iou, best_iou)

    am = jnp.full((_S, _L), -1, jnp.int32)
    for g in range(_G):
        mg = jnp.max(ious[g])
        bai = jnp.min(jnp.where(ious[g] == mg, aid, _AP))
        am = jnp.where(aid == bai, g, am)
    am = jnp.where(best_iou >= _IOU_THR, best_g, am)
    pos = am >= 0
    mi = jnp.where(pos, am, 0)

    clsv = jnp.zeros((_S, _L), jnp.int32)
    zf = jnp.zeros((_S, _L), jnp.float32)
    gcxv, gcyv, gwv, ghv = zf, zf, zf, zf
    lmkv = [zf] * 10
    gz0v, gz1v = zf, zf
    for g in range(_G):
        m = mi == g
        clsv = jnp.where(m, gtl[0, 0, g], clsv)
        gx1 = gtb[0, g, 0]
        gy1 = gtb[0, g, 1]
        gx2 = gtb[0, g, 2]
        gy2 = gtb[0, g, 3]
        gcxv = jnp.where(m, (gx1 + gx2) / 2.0, gcxv)
        gcyv = jnp.where(m, (gy1 + gy2) / 2.0, gcyv)
        gwv = jnp.where(m, gx2 - gx1, gwv)
        ghv = jnp.where(m, gy2 - gy1, ghv)
        for i in range(10):
            lmkv[i] = jnp.where(m, gtlmk[0, g, i], lmkv[i])
        gz0v = jnp.where(m, gtgz[0, g, 0], gz0v)
        gz1v = jnp.where(m, gtgz[0, g, 1], gz1v)

    classes = jnp.where(pos, clsv, 0)
    face = pos & (classes == 1)

    tx = (gcxv - dcx) / (dw * 0.1)
    ty = (gcyv - dcy) / (dh * 0.1)
    tw = jnp.log(jnp.maximum(gwv / dw, 1e-6)) / 0.2
    th = jnp.log(jnp.maximum(ghv / dh, 1e-6)) / 0.2
    box_v = (_smooth_l1(box_ref[0, 0] - tx) + _smooth_l1(box_ref[0, 1] - ty)
             + _smooth_l1(box_ref[0, 2] - tw) + _smooth_l1(box_ref[0, 3] - th))
    box_sum = jnp.sum(jnp.where(pos, box_v, 0.0))

    lmk_v = zf
    for i in range(5):
        ox = (lmkv[2 * i] - dcx) / (dw * 0.1)
        oy = (lmkv[2 * i + 1] - dcy) / (dh * 0.1)
        lmk_v = lmk_v + _smooth_l1(lmk_ref[0, 2 * i] - ox)
        lmk_v = lmk_v + _smooth_l1(lmk_ref[0, 2 * i + 1] - oy)
    lmk_sum = jnp.sum(jnp.where(face, lmk_v, 0.0))

    gaze_v = _smooth_l1(gaze_ref[0, 0] - gz0v) + _smooth_l1(gaze_ref[0, 1] - gz1v)
    gaze_sum = jnp.sum(jnp.where(face, gaze_v, 0.0))

    l0 = cls_ref[0, 0]
    l1 = cls_ref[0, 1]
    l2 = cls_ref[0, 2]
    mx = jnp.maximum(jnp.maximum(l0, l1), l2)
    lse = mx + jnp.log(jnp.exp(l0 - mx) + jnp.exp(l1 - mx) + jnp.exp(l2 - mx))
    lsel = jnp.where(classes == 1, l1, jnp.where(classes == 2, l2, l0))
    ce = lse - lsel

    np_i = jnp.sum(jnp.where(pos, 1, 0))
    nf_i = jnp.sum(jnp.where(face, 1, 0))
    cls_pos = jnp.sum(jnp.where(pos, ce, 0.0))
    k = jnp.minimum(_A - np_i, _NEG_POS_RATIO * jnp.maximum(np_i, 1))

    si = jax.lax.bitcast_convert_type(ce, jnp.int32)
    s = jnp.where(si >= 0, si, si ^ _I32_MAXP)
    s = jnp.where(valid & (~pos), s, _I32_MIN)
    p = jnp.int32(0)
    for bit in range(31, -1, -1):
        if bit == 31:
            cand = _I32_MIN
        else:
            cand = p | jnp.int32(1 << bit)
        cand_s = cand ^ _I32_MIN
        cnt = jnp.sum(jnp.where(s >= cand_s, 1, 0))
        p = jnp.where(cnt >= k, cand, p)
    ts = p ^ _I32_MIN
    ti = jnp.where(ts >= 0, ts, ts ^ _I32_MAXP)
    tf = jax.lax.bitcast_convert_type(ti, jnp.float32)
    cnt_gt = jnp.sum(jnp.where(s > ts, 1, 0))
    sum_gt = jnp.sum(jnp.where(s > ts, ce, 0.0))
    neg_sum = jnp.where(k > 0,
                        sum_gt + (k - cnt_gt).astype(jnp.float32) * tf, 0.0)

    acc[0] = acc[0] + cls_pos + neg_sum
    acc[1] = acc[1] + box_sum
    acc[2] = acc[2] + lmk_sum
    acc[3] = acc[3] + gaze_sum
    acc[4] = acc[4] + jnp.maximum(np_i.astype(jnp.float32), 1.0)
    acc[5] = acc[5] + jnp.maximum(nf_i.astype(jnp.float32), 1.0)

    @pl.when(b == _B - 1)
    def _fin():
        tcls = acc[0] / acc[4]
        tbox = acc[1] / acc[4]
        tlmk = acc[2] / acc[5]
        tgaze = acc[3] / acc[5]
        out_ref[0] = tcls + tbox + 0.5 * tlmk + 0.5 * tgaze
        out_ref[1] = tcls
        out_ref[2] = tbox
        out_ref[3] = tlmk
        out_ref[4] = tgaze


def _prep(x):
    k = x.shape[-1]
    xt = jnp.transpose(x, (0, 2, 1))
    xt = jnp.pad(xt, ((0, 0), (0, 0), (0, _AP - _A)))
    return xt.reshape(_B, k, _S, _L)


def kernel(cls_logits, box_preds, lmk_preds, gaze_preds, default_boxes,
           gt_boxes, gt_landmarks, gt_gaze, gt_labels):
    clsT = _prep(cls_logits)
    boxT = _prep(box_preds)
    lmkT = _prep(lmk_preds)
    gazeT = _prep(gaze_preds)
    dbT = jnp.pad(default_boxes.T, ((0, 0), (0, _AP - _A))).reshape(4, _S, _L)
    gtl = gt_labels.astype(jnp.int32).reshape(_B, 1, _G)

    out = pl.pallas_call(
        _body,
        grid=(_B,),
        in_specs=[
            pl.BlockSpec((1, _C, _S, _L), lambda b: (b, 0, 0, 0)),
            pl.BlockSpec((1, 4, _S, _L), lambda b: (b, 0, 0, 0)),
            pl.BlockSpec((1, 10, _S, _L), lambda b: (b, 0, 0, 0)),
            pl.BlockSpec((1, 2, _S, _L), lambda b: (b, 0, 0, 0)),
            pl.BlockSpec((4, _S, _L), lambda b: (0, 0, 0)),
            pl.BlockSpec((1, _G, 4), lambda b: (b, 0, 0),
                         memory_space=pltpu.SMEM),
            pl.BlockSpec((1, 1, _G), lambda b: (b, 0, 0),
                         memory_space=pltpu.SMEM),
            pl.BlockSpec((1, _G, 10), lambda b: (b, 0, 0),
                         memory_space=pltpu.SMEM),
            pl.BlockSpec((1, _G, 2), lambda b: (b, 0, 0),
                         memory_space=pltpu.SMEM),
        ],
        out_specs=pl.BlockSpec((5,), lambda b: (0,),
                               memory_space=pltpu.SMEM),
        out_shape=jax.ShapeDtypeStruct((5,), jnp.float32),
        scratch_shapes=[pltpu.SMEM((8,), jnp.float32)],
    )(clsT, boxT, lmkT, gazeT, dbT, gt_boxes, gtl, gt_landmarks, gt_gaze)
    return out


# speedup vs baseline: 64.5278x; 1.0660x over previous
import jax
import jax.numpy as jnp
from jax.experimental import pallas as pl
from jax.experimental.pallas import tpu as pltpu

_B, _A, _G, _C = 16, 24564, 8, 3
_AP = 24576
_S, _L = 192, 128
_NEG_POS_RATIO = 3
_IOU_THR = 0.5
_I32_MIN = -2147483648
_I32_MAXP = 2147483647


def _smooth_l1(x):
    a = jnp.abs(x)
    return jnp.where(a < 1.0, 0.5 * a * a, a - 0.5)


def _body(cls_ref, box_ref, lmk_ref, gaze_ref, db_ref, gtb, gtl, gtlmk, gtgz,
          out_ref, acc, s_all):
    b = pl.program_id(0)

    @pl.when(b == 0)
    def _init():
        for i in range(6):
            acc[i] = 0.0

    dcx = db_ref[0]
    dcy = db_ref[1]
    dw = db_ref[2]
    dh = db_ref[3]
    dx1 = dcx - dw / 2.0
    dy1 = dcy - dh / 2.0
    dx2 = dcx + dw / 2.0
    dy2 = dcy + dh / 2.0
    area_b = (dx2 - dx1) * (dy2 - dy1)

    sl = jax.lax.broadcasted_iota(jnp.int32, (_S, _L), 0)
    ll = jax.lax.broadcasted_iota(jnp.int32, (_S, _L), 1)
    aid = sl * _L + ll
    valid = aid < _A

    ious = []
    best_iou = None
    best_g = None
    for g in range(_G):
        gx1 = gtb[0, g, 0]
        gy1 = gtb[0, g, 1]
        gx2 = gtb[0, g, 2]
        gy2 = gtb[0, g, 3]
        w = jnp.maximum(jnp.minimum(gx2, dx2) - jnp.maximum(gx1, dx1), 0.0)
        h = jnp.maximum(jnp.minimum(gy2, dy2) - jnp.maximum(gy1, dy1), 0.0)
        inter = w * h
        area_a = (gx2 - gx1) * (gy2 - gy1)
        union = jnp.maximum(area_a + area_b - inter, 1e-8)
        iou = inter / union
        ious.append(iou)
        if g == 0:
            best_iou = iou
            best_g = jnp.zeros((_S, _L), jnp.int32)
        else:
            upd = iou > best_iou
            best_g = jnp.where(upd, g, best_g)
            best_iou = jnp.where(upd, iou, best_iou)

    am = jnp.full((_S, _L), -1, jnp.int32)
    for g in range(_G):
        mg = jnp.max(ious[g])
        bai = jnp.min(jnp.where(ious[g] == mg, aid, _AP))
        am = jnp.where(aid == bai, g, am)
    am = jnp.where(best_iou >= _IOU_THR, best_g, am)
    pos = am >= 0
    mi = jnp.where(pos, am, 0)

    clsv = jnp.zeros((_S, _L), jnp.int32)
    zf = jnp.zeros((_S, _L), jnp.float32)
    gcxv, gcyv, gwv, ghv = zf, zf, zf, zf
    lmkv = [zf] * 10
    gz0v, gz1v = zf, zf
    for g in range(_G):
        m = mi == g
        clsv = jnp.where(m, gtl[0, 0, g], clsv)
        gx1 = gtb[0, g, 0]
        gy1 = gtb[0, g, 1]
        gx2 = gtb[0, g, 2]
        gy2 = gtb[0, g, 3]
        gcxv = jnp.where(m, (gx1 + gx2) / 2.0, gcxv)
        gcyv = jnp.where(m, (gy1 + gy2) / 2.0, gcyv)
        gwv = jnp.where(m, gx2 - gx1, gwv)
        ghv = jnp.where(m, gy2 - gy1, ghv)
        for i in range(10):
            lmkv[i] = jnp.where(m, gtlmk[0, g, i], lmkv[i])
        gz0v = jnp.where(m, gtgz[0, g, 0], gz0v)
        gz1v = jnp.where(m, gtgz[0, g, 1], gz1v)

    classes = jnp.where(pos, clsv, 0)
    face = pos & (classes == 1)

    tx = (gcxv - dcx) / (dw * 0.1)
    ty = (gcyv - dcy) / (dh * 0.1)
    tw = jnp.log(jnp.maximum(gwv / dw, 1e-6)) / 0.2
    th = jnp.log(jnp.maximum(ghv / dh, 1e-6)) / 0.2
    box_v = (_smooth_l1(box_ref[0, 0] - tx) + _smooth_l1(box_ref[0, 1] - ty)
             + _smooth_l1(box_ref[0, 2] - tw) + _smooth_l1(box_ref[0, 3] - th))
    box_sum = jnp.sum(jnp.where(pos, box_v, 0.0))

    lmk_v = zf
    for i in range(5):
        ox = (lmkv[2 * i] - dcx) / (dw * 0.1)
        oy = (lmkv[2 * i + 1] - dcy) / (dh * 0.1)
        lmk_v = lmk_v + _smooth_l1(lmk_ref[0, 2 * i] - ox)
        lmk_v = lmk_v + _smooth_l1(lmk_ref[0, 2 * i + 1] - oy)
    lmk_sum = jnp.sum(jnp.where(face, lmk_v, 0.0))

    gaze_v = _smooth_l1(gaze_ref[0, 0] - gz0v) + _smooth_l1(gaze_ref[0, 1] - gz1v)
    gaze_sum = jnp.sum(jnp.where(face, gaze_v, 0.0))

    l0 = cls_ref[0, 0]
    l1 = cls_ref[0, 1]
    l2 = cls_ref[0, 2]
    mx = jnp.maximum(jnp.maximum(l0, l1), l2)
    lse = mx + jnp.log(jnp.exp(l0 - mx) + jnp.exp(l1 - mx) + jnp.exp(l2 - mx))
    lsel = jnp.where(classes == 1, l1, jnp.where(classes == 2, l2, l0))
    ce = lse - lsel

    nf_i = jnp.sum(jnp.where(face, 1, 0))
    cls_pos = jnp.sum(jnp.where(pos, ce, 0.0))

    si = jax.lax.bitcast_convert_type(ce, jnp.int32)
    s = jnp.where(si >= 0, si, si ^ _I32_MAXP)
    s = jnp.where(valid & (~pos), s, _I32_MIN)
    s_all[pl.ds(b, 1)] = s[None]

    acc[0] = acc[0] + cls_pos
    acc[1] = acc[1] + box_sum
    acc[2] = acc[2] + lmk_sum
    acc[3] = acc[3] + gaze_sum
    acc[5] = acc[5] + jnp.maximum(nf_i.astype(jnp.float32), 1.0)

    @pl.when(b == _B - 1)
    def _fin():
        sv = s_all[:]

        def _red(x):
            return jnp.sum(jnp.sum(x, axis=2), axis=1)

        n_min = _red(jnp.where(sv == _I32_MIN, 1, 0))
        np_vec = n_min - (_AP - _A)
        k_vec = jnp.minimum(_A - np_vec,
                            _NEG_POS_RATIO * jnp.maximum(np_vec, 1))
        p = jnp.zeros((_B,), jnp.int32)
        for bit in range(31, -1, -1):
            if bit == 31:
                cand = jnp.full((_B,), _I32_MIN, jnp.int32)
            else:
                cand = p | jnp.int32(1 << bit)
            cand_s = (cand ^ _I32_MIN).reshape(_B, 1, 1)
            cnt = _red(jnp.where(sv >= cand_s, 1, 0))
            p = jnp.where(cnt >= k_vec, cand, p)
        ts = p ^ _I32_MIN
        ti = jnp.where(ts >= 0, ts, ts ^ _I32_MAXP)
        tf = jax.lax.bitcast_convert_type(ti, jnp.float32)
        tsb = ts.reshape(_B, 1, 1)
        gt_m = sv > tsb
        cnt_gt = _red(jnp.where(gt_m, 1, 0))
        cei = jnp.where(sv >= 0, sv, sv ^ _I32_MAXP)
        cef = jax.lax.bitcast_convert_type(cei, jnp.float32)
        sum_gt = jnp.sum(jnp.sum(jnp.where(gt_m, cef, 0.0), axis=2), axis=1)
        neg_vec = jnp.where(k_vec > 0,
                            sum_gt + (k_vec - cnt_gt).astype(jnp.float32) * tf,
                            0.0)
        neg_total = jnp.sum(neg_vec)
        npf_total = jnp.sum(jnp.maximum(np_vec, 1).astype(jnp.float32))

        tcls = (acc[0] + neg_total) / npf_total
        tbox = acc[1] / npf_total
        tlmk = acc[2] / acc[5]
        tgaze = acc[3] / acc[5]
        out_ref[0] = tcls + tbox + 0.5 * tlmk + 0.5 * tgaze
        out_ref[1] = tcls
        out_ref[2] = tbox
        out_ref[3] = tlmk
        out_ref[4] = tgaze


def _prep(x):
    k = x.shape[-1]
    xt = jnp.transpose(x, (0, 2, 1))
    xt = jnp.pad(xt, ((0, 0), (0, 0), (0, _AP - _A)))
    return xt.reshape(_B, k, _S, _L)


def kernel(cls_logits, box_preds, lmk_preds, gaze_preds, default_boxes,
           gt_boxes, gt_landmarks, gt_gaze, gt_labels):
    clsT = _prep(cls_logits)
    boxT = _prep(box_preds)
    lmkT = _prep(lmk_preds)
    gazeT = _prep(gaze_preds)
    dbT = jnp.pad(default_boxes.T, ((0, 0), (0, _AP - _A))).reshape(4, _S, _L)
    gtl = gt_labels.astype(jnp.int32).reshape(_B, 1, _G)

    out = pl.pallas_call(
        _body,
        grid=(_B,),
        in_specs=[
            pl.BlockSpec((1, _C, _S, _L), lambda b: (b, 0, 0, 0)),
            pl.BlockSpec((1, 4, _S, _L), lambda b: (b, 0, 0, 0)),
            pl.BlockSpec((1, 10, _S, _L), lambda b: (b, 0, 0, 0)),
            pl.BlockSpec((1, 2, _S, _L), lambda b: (b, 0, 0, 0)),
            pl.BlockSpec((4, _S, _L), lambda b: (0, 0, 0)),
            pl.BlockSpec((1, _G, 4), lambda b: (b, 0, 0),
                         memory_space=pltpu.SMEM),
            pl.BlockSpec((1, 1, _G), lambda b: (b, 0, 0),
                         memory_space=pltpu.SMEM),
            pl.BlockSpec((1, _G, 10), lambda b: (b, 0, 0),
                         memory_space=pltpu.SMEM),
            pl.BlockSpec((1, _G, 2), lambda b: (b, 0, 0),
                         memory_space=pltpu.SMEM),
        ],
        out_specs=pl.BlockSpec((5,), lambda b: (0,),
                               memory_space=pltpu.SMEM),
        out_shape=jax.ShapeDtypeStruct((5,), jnp.float32),
        scratch_shapes=[pltpu.SMEM((8,), jnp.float32),
                        pltpu.VMEM((_B, _S, _L), jnp.int32)],
    )(clsT, boxT, lmkT, gazeT, dbT, gt_boxes, gtl, gt_landmarks, gt_gaze)
    return out


# speedup vs baseline: 83.7378x; 1.2977x over previous
import jax
import jax.numpy as jnp
from jax.experimental import pallas as pl
from jax.experimental.pallas import tpu as pltpu

_B, _A, _G, _C = 16, 24564, 8, 3
_AP = 24576
_S, _L = 192, 128
_NEG_POS_RATIO = 3
_IOU_THR = 0.5
_I32_MIN = -2147483648
_I32_MAXP = 2147483647


def _smooth_l1(x):
    a = jnp.abs(x)
    return jnp.where(a < 1.0, 0.5 * a * a, a - 0.5)


def _body(cls_ref, box_ref, lmk_ref, gaze_ref, db_ref, gtb, gtl, gtlmk, gtgz,
          out_ref, acc, s_all):
    b = pl.program_id(0)

    @pl.when(b == 0)
    def _init():
        for i in range(6):
            acc[i] = 0.0

    dcx = db_ref[0]
    dcy = db_ref[1]
    dw = db_ref[2]
    dh = db_ref[3]
    dx1 = dcx - dw / 2.0
    dy1 = dcy - dh / 2.0
    dx2 = dcx + dw / 2.0
    dy2 = dcy + dh / 2.0
    area_b = (dx2 - dx1) * (dy2 - dy1)

    sl = jax.lax.broadcasted_iota(jnp.int32, (_S, _L), 0)
    ll = jax.lax.broadcasted_iota(jnp.int32, (_S, _L), 1)
    aid = sl * _L + ll
    valid = aid < _A

    ious = []
    best_iou = None
    best_g = None
    for g in range(_G):
        gx1 = gtb[0, g, 0]
        gy1 = gtb[0, g, 1]
        gx2 = gtb[0, g, 2]
        gy2 = gtb[0, g, 3]
        w = jnp.maximum(jnp.minimum(gx2, dx2) - jnp.maximum(gx1, dx1), 0.0)
        h = jnp.maximum(jnp.minimum(gy2, dy2) - jnp.maximum(gy1, dy1), 0.0)
        inter = w * h
        area_a = (gx2 - gx1) * (gy2 - gy1)
        union = jnp.maximum(area_a + area_b - inter, 1e-8)
        iou = inter / union
        ious.append(iou)
        if g == 0:
            best_iou = iou
            best_g = jnp.zeros((_S, _L), jnp.int32)
        else:
            upd = iou > best_iou
            best_g = jnp.where(upd, g, best_g)
            best_iou = jnp.where(upd, iou, best_iou)

    am = jnp.full((_S, _L), -1, jnp.int32)
    for g in range(_G):
        mg = jnp.max(jnp.max(ious[g], axis=0))
        bai = jnp.min(jnp.min(jnp.where(ious[g] == mg, aid, _AP), axis=0))
        am = jnp.where(aid == bai, g, am)
    am = jnp.where(best_iou >= _IOU_THR, best_g, am)
    pos = am >= 0
    mi = jnp.where(pos, am, 0)

    clsv = jnp.zeros((_S, _L), jnp.int32)
    zf = jnp.zeros((_S, _L), jnp.float32)
    gcxv, gcyv, gwv, ghv = zf, zf, zf, zf
    lmkv = [zf] * 10
    gz0v, gz1v = zf, zf
    for g in range(_G):
        m = mi == g
        clsv = jnp.where(m, gtl[0, 0, g], clsv)
        gx1 = gtb[0, g, 0]
        gy1 = gtb[0, g, 1]
        gx2 = gtb[0, g, 2]
        gy2 = gtb[0, g, 3]
        gcxv = jnp.where(m, (gx1 + gx2) / 2.0, gcxv)
        gcyv = jnp.where(m, (gy1 + gy2) / 2.0, gcyv)
        gwv = jnp.where(m, gx2 - gx1, gwv)
        ghv = jnp.where(m, gy2 - gy1, ghv)
        for i in range(10):
            lmkv[i] = jnp.where(m, gtlmk[0, g, i], lmkv[i])
        gz0v = jnp.where(m, gtgz[0, g, 0], gz0v)
        gz1v = jnp.where(m, gtgz[0, g, 1], gz1v)

    classes = jnp.where(pos, clsv, 0)
    face = pos & (classes == 1)

    tx = (gcxv - dcx) / (dw * 0.1)
    ty = (gcyv - dcy) / (dh * 0.1)
    tw = jnp.log(jnp.maximum(gwv / dw, 1e-6)) / 0.2
    th = jnp.log(jnp.maximum(ghv / dh, 1e-6)) / 0.2
    box_v = (_smooth_l1(box_ref[0, 0] - tx) + _smooth_l1(box_ref[0, 1] - ty)
             + _smooth_l1(box_ref[0, 2] - tw) + _smooth_l1(box_ref[0, 3] - th))
    box_sum = jnp.sum(jnp.where(pos, box_v, 0.0))

    lmk_v = zf
    for i in range(5):
        ox = (lmkv[2 * i] - dcx) / (dw * 0.1)
        oy = (lmkv[2 * i + 1] - dcy) / (dh * 0.1)
        lmk_v = lmk_v + _smooth_l1(lmk_ref[0, 2 * i] - ox)
        lmk_v = lmk_v + _smooth_l1(lmk_ref[0, 2 * i + 1] - oy)
    lmk_sum = jnp.sum(jnp.where(face, lmk_v, 0.0))

    gaze_v = _smooth_l1(gaze_ref[0, 0] - gz0v) + _smooth_l1(gaze_ref[0, 1] - gz1v)
    gaze_sum = jnp.sum(jnp.where(face, gaze_v, 0.0))

    l0 = cls_ref[0, 0]
    l1 = cls_ref[0, 1]
    l2 = cls_ref[0, 2]
    mx = jnp.maximum(jnp.maximum(l0, l1), l2)
    lse = mx + jnp.log(jnp.exp(l0 - mx) + jnp.exp(l1 - mx) + jnp.exp(l2 - mx))
    lsel = jnp.where(classes == 1, l1, jnp.where(classes == 2, l2, l0))
    ce = lse - lsel

    nf_i = jnp.sum(jnp.where(face, 1, 0))
    cls_pos = jnp.sum(jnp.where(pos, ce, 0.0))

    si = jax.lax.bitcast_convert_type(ce, jnp.int32)
    s = jnp.where(si >= 0, si, si ^ _I32_MAXP)
    s = jnp.where(valid & (~pos), s, _I32_MIN)
    s_all[pl.ds(b, 1)] = s[None]

    acc[0] = acc[0] + cls_pos
    acc[1] = acc[1] + box_sum
    acc[2] = acc[2] + lmk_sum
    acc[3] = acc[3] + gaze_sum
    acc[5] = acc[5] + jnp.maximum(nf_i.astype(jnp.float32), 1.0)

    @pl.when(b == _B - 1)
    def _fin():
        sv = s_all[:]

        def _red(x):
            return jnp.sum(jnp.sum(x, axis=1), axis=1)

        n_min = _red(jnp.where(sv == _I32_MIN, 1, 0))
        np_vec = n_min - (_AP - _A)
        k_vec = jnp.minimum(_A - np_vec,
                            _NEG_POS_RATIO * jnp.maximum(np_vec, 1))
        p = jnp.full((_B,), _I32_MIN, jnp.int32)
        for bit in range(30, -1, -1):
            cand = p | jnp.int32(1 << bit)
            cand_s = (cand ^ _I32_MIN).reshape(_B, 1, 1)
            cnt = _red(jnp.where(sv >= cand_s, 1, 0))
            p = jnp.where(cnt >= k_vec, cand, p)
        ts = p ^ _I32_MIN
        ti = jnp.where(ts >= 0, ts, ts ^ _I32_MAXP)
        tf = jax.lax.bitcast_convert_type(ti, jnp.float32)
        tsb = ts.reshape(_B, 1, 1)
        gt_m = sv > tsb
        cnt_gt = _red(jnp.where(gt_m, 1, 0))
        cei = jnp.where(sv >= 0, sv, sv ^ _I32_MAXP)
        cef = jax.lax.bitcast_convert_type(cei, jnp.float32)
        sum_gt = jnp.sum(jnp.sum(jnp.where(gt_m, cef, 0.0), axis=1), axis=1)
        neg_vec = jnp.where(k_vec > 0,
                            sum_gt + (k_vec - cnt_gt).astype(jnp.float32) * tf,
                            0.0)
        neg_total = jnp.sum(neg_vec)
        npf_total = jnp.sum(jnp.maximum(np_vec, 1).astype(jnp.float32))

        tcls = (acc[0] + neg_total) / npf_total
        tbox = acc[1] / npf_total
        tlmk = acc[2] / acc[5]
        tgaze = acc[3] / acc[5]
        out_ref[0] = tcls + tbox + 0.5 * tlmk + 0.5 * tgaze
        out_ref[1] = tcls
        out_ref[2] = tbox
        out_ref[3] = tlmk
        out_ref[4] = tgaze


def _prep(x):
    k = x.shape[-1]
    xt = jnp.transpose(x, (0, 2, 1))
    xt = jnp.pad(xt, ((0, 0), (0, 0), (0, _AP - _A)))
    return xt.reshape(_B, k, _S, _L)


def kernel(cls_logits, box_preds, lmk_preds, gaze_preds, default_boxes,
           gt_boxes, gt_landmarks, gt_gaze, gt_labels):
    clsT = _prep(cls_logits)
    boxT = _prep(box_preds)
    lmkT = _prep(lmk_preds)
    gazeT = _prep(gaze_preds)
    dbT = jnp.pad(default_boxes.T, ((0, 0), (0, _AP - _A))).reshape(4, _S, _L)
    gtl = gt_labels.astype(jnp.int32).reshape(_B, 1, _G)

    out = pl.pallas_call(
        _body,
        grid=(_B,),
        in_specs=[
            pl.BlockSpec((1, _C, _S, _L), lambda b: (b, 0, 0, 0)),
            pl.BlockSpec((1, 4, _S, _L), lambda b: (b, 0, 0, 0)),
            pl.BlockSpec((1, 10, _S, _L), lambda b: (b, 0, 0, 0)),
            pl.BlockSpec((1, 2, _S, _L), lambda b: (b, 0, 0, 0)),
            pl.BlockSpec((4, _S, _L), lambda b: (0, 0, 0)),
            pl.BlockSpec((1, _G, 4), lambda b: (b, 0, 0),
                         memory_space=pltpu.SMEM),
            pl.BlockSpec((1, 1, _G), lambda b: (b, 0, 0),
                         memory_space=pltpu.SMEM),
            pl.BlockSpec((1, _G, 10), lambda b: (b, 0, 0),
                         memory_space=pltpu.SMEM),
            pl.BlockSpec((1, _G, 2), lambda b: (b, 0, 0),
                         memory_space=pltpu.SMEM),
        ],
        out_specs=pl.BlockSpec((5,), lambda b: (0,),
                               memory_space=pltpu.SMEM),
        out_shape=jax.ShapeDtypeStruct((5,), jnp.float32),
        scratch_shapes=[pltpu.SMEM((8,), jnp.float32),
                        pltpu.VMEM((_B, _S, _L), jnp.int32)],
    )(clsT, boxT, lmkT, gazeT, dbT, gt_boxes, gtl, gt_landmarks, gt_gaze)
    return out


# speedup vs baseline: 91.8485x; 1.0969x over previous
import jax
import jax.numpy as jnp
from jax.experimental import pallas as pl
from jax.experimental.pallas import tpu as pltpu

_B, _A, _G, _C = 16, 24564, 8, 3
_AP = 24576
_S, _L = 192, 128
_NEG_POS_RATIO = 3
_IOU_THR = 0.5
_I32_MIN = -2147483648
_I32_MAXP = 2147483647


def _smooth_l1(x):
    a = jnp.abs(x)
    return jnp.where(a < 1.0, 0.5 * a * a, a - 0.5)


def _body(cls_ref, box_ref, lmk_ref, gaze_ref, db_ref, gtb, gtl, gtlmk, gtgz,
          out_ref, acc, s_all):
    b = pl.program_id(0)

    @pl.when(b == 0)
    def _init():
        for i in range(6):
            acc[i] = 0.0

    dcx = db_ref[0]
    dcy = db_ref[1]
    dw = db_ref[2]
    dh = db_ref[3]
    dx1 = dcx - dw / 2.0
    dy1 = dcy - dh / 2.0
    dx2 = dcx + dw / 2.0
    dy2 = dcy + dh / 2.0
    area_b = (dx2 - dx1) * (dy2 - dy1)

    sl = jax.lax.broadcasted_iota(jnp.int32, (_S, _L), 0)
    ll = jax.lax.broadcasted_iota(jnp.int32, (_S, _L), 1)
    aid = sl * _L + ll
    valid = aid < _A

    ious = []
    best_iou = None
    best_g = None
    for g in range(_G):
        gx1 = gtb[0, g, 0]
        gy1 = gtb[0, g, 1]
        gx2 = gtb[0, g, 2]
        gy2 = gtb[0, g, 3]
        w = jnp.maximum(jnp.minimum(gx2, dx2) - jnp.maximum(gx1, dx1), 0.0)
        h = jnp.maximum(jnp.minimum(gy2, dy2) - jnp.maximum(gy1, dy1), 0.0)
        inter = w * h
        area_a = (gx2 - gx1) * (gy2 - gy1)
        union = jnp.maximum(area_a + area_b - inter, 1e-8)
        iou = inter / union
        ious.append(iou)
        if g == 0:
            best_iou = iou
            best_g = jnp.zeros((_S, _L), jnp.int32)
        else:
            upd = iou > best_iou
            best_g = jnp.where(upd, g, best_g)
            best_iou = jnp.where(upd, iou, best_iou)

    am = jnp.full((_S, _L), -1, jnp.int32)
    for g in range(_G):
        mg = jnp.max(jnp.max(ious[g], axis=0))
        bai = jnp.min(jnp.min(jnp.where(ious[g] == mg, aid, _AP), axis=0))
        am = jnp.where(aid == bai, g, am)
    am = jnp.where(best_iou >= _IOU_THR, best_g, am)
    pos = am >= 0
    mi = jnp.where(pos, am, 0)

    clsv = jnp.zeros((_S, _L), jnp.int32)
    zf = jnp.zeros((_S, _L), jnp.float32)
    gcxv, gcyv, gwv, ghv = zf, zf, zf, zf
    lmkv = [zf] * 10
    gz0v, gz1v = zf, zf
    for g in range(_G):
        m = mi == g
        clsv = jnp.where(m, gtl[0, 0, g], clsv)
        gx1 = gtb[0, g, 0]
        gy1 = gtb[0, g, 1]
        gx2 = gtb[0, g, 2]
        gy2 = gtb[0, g, 3]
        gcxv = jnp.where(m, (gx1 + gx2) / 2.0, gcxv)
        gcyv = jnp.where(m, (gy1 + gy2) / 2.0, gcyv)
        gwv = jnp.where(m, gx2 - gx1, gwv)
        ghv = jnp.where(m, gy2 - gy1, ghv)
        for i in range(10):
            lmkv[i] = jnp.where(m, gtlmk[0, g, i], lmkv[i])
        gz0v = jnp.where(m, gtgz[0, g, 0], gz0v)
        gz1v = jnp.where(m, gtgz[0, g, 1], gz1v)

    classes = jnp.where(pos, clsv, 0)
    face = pos & (classes == 1)

    tx = (gcxv - dcx) / (dw * 0.1)
    ty = (gcyv - dcy) / (dh * 0.1)
    tw = jnp.log(jnp.maximum(gwv / dw, 1e-6)) / 0.2
    th = jnp.log(jnp.maximum(ghv / dh, 1e-6)) / 0.2
    bp = [box_ref[0, j].astype(jnp.float32) for j in range(4)]
    box_v = (_smooth_l1(bp[0] - tx) + _smooth_l1(bp[1] - ty)
             + _smooth_l1(bp[2] - tw) + _smooth_l1(bp[3] - th))
    box_sum = jnp.sum(jnp.where(pos, box_v, 0.0))

    lmk_v = zf
    for i in range(5):
        ox = (lmkv[2 * i] - dcx) / (dw * 0.1)
        oy = (lmkv[2 * i + 1] - dcy) / (dh * 0.1)
        lmk_v = lmk_v + _smooth_l1(lmk_ref[0, 2 * i].astype(jnp.float32) - ox)
        lmk_v = lmk_v + _smooth_l1(
            lmk_ref[0, 2 * i + 1].astype(jnp.float32) - oy)
    lmk_sum = jnp.sum(jnp.where(face, lmk_v, 0.0))

    gaze_v = (_smooth_l1(gaze_ref[0, 0].astype(jnp.float32) - gz0v)
              + _smooth_l1(gaze_ref[0, 1].astype(jnp.float32) - gz1v))
    gaze_sum = jnp.sum(jnp.where(face, gaze_v, 0.0))

    l0 = cls_ref[0, 0]
    l1 = cls_ref[0, 1]
    l2 = cls_ref[0, 2]
    mx = jnp.maximum(jnp.maximum(l0, l1), l2)
    lse = mx + jnp.log(jnp.exp(l0 - mx) + jnp.exp(l1 - mx) + jnp.exp(l2 - mx))
    lsel = jnp.where(classes == 1, l1, jnp.where(classes == 2, l2, l0))
    ce = lse - lsel

    nf_i = jnp.sum(jnp.where(face, 1, 0))
    cls_pos = jnp.sum(jnp.where(pos, ce, 0.0))

    si = jax.lax.bitcast_convert_type(ce, jnp.int32)
    s = jnp.where(si >= 0, si, si ^ _I32_MAXP)
    s = jnp.where(valid & (~pos), s, _I32_MIN)
    s_all[pl.ds(b, 1)] = s[None]

    acc[0] = acc[0] + cls_pos
    acc[1] = acc[1] + box_sum
    acc[2] = acc[2] + lmk_sum
    acc[3] = acc[3] + gaze_sum
    acc[5] = acc[5] + jnp.maximum(nf_i.astype(jnp.float32), 1.0)

    @pl.when(b == _B - 1)
    def _fin():
        sv = s_all[:]

        def _red(x):
            return jnp.sum(jnp.sum(x, axis=1), axis=1)

        n_min = _red(jnp.where(sv == _I32_MIN, 1, 0))
        np_vec = n_min - (_AP - _A)
        k_vec = jnp.minimum(_A - np_vec,
                            _NEG_POS_RATIO * jnp.maximum(np_vec, 1))
        p = jnp.full((_B,), _I32_MIN, jnp.int32)
        for bit in range(30, -1, -1):
            cand = p | jnp.int32(1 << bit)
            cand_s = (cand ^ _I32_MIN).reshape(_B, 1, 1)
            cnt = _red(jnp.where(sv >= cand_s, 1, 0))
            p = jnp.where(cnt >= k_vec, cand, p)
        ts = p ^ _I32_MIN
        ti = jnp.where(ts >= 0, ts, ts ^ _I32_MAXP)
        tf = jax.lax.bitcast_convert_type(ti, jnp.float32)
        tsb = ts.reshape(_B, 1, 1)
        gt_m = sv > tsb
        cnt_gt = _red(jnp.where(gt_m, 1, 0))
        cei = jnp.where(sv >= 0, sv, sv ^ _I32_MAXP)
        cef = jax.lax.bitcast_convert_type(cei, jnp.float32)
        sum_gt = jnp.sum(jnp.sum(jnp.where(gt_m, cef, 0.0), axis=1), axis=1)
        neg_vec = jnp.where(k_vec > 0,
                            sum_gt + (k_vec - cnt_gt).astype(jnp.float32) * tf,
                            0.0)
        neg_total = jnp.sum(neg_vec)
        npf_total = jnp.sum(jnp.maximum(np_vec, 1).astype(jnp.float32))

        tcls = (acc[0] + neg_total) / npf_total
        tbox = acc[1] / npf_total
        tlmk = acc[2] / acc[5]
        tgaze = acc[3] / acc[5]
        out_ref[0] = tcls + tbox + 0.5 * tlmk + 0.5 * tgaze
        out_ref[1] = tcls
        out_ref[2] = tbox
        out_ref[3] = tlmk
        out_ref[4] = tgaze


def _prep(x, dtype=jnp.float32):
    k = x.shape[-1]
    xt = jnp.transpose(x, (0, 2, 1)).astype(dtype)
    xt = jnp.pad(xt, ((0, 0), (0, 0), (0, _AP - _A)))
    return xt.reshape(_B, k, _S, _L)


def kernel(cls_logits, box_preds, lmk_preds, gaze_preds, default_boxes,
           gt_boxes, gt_landmarks, gt_gaze, gt_labels):
    clsT = _prep(cls_logits)
    boxT = _prep(box_preds, jnp.bfloat16)
    lmkT = _prep(lmk_preds, jnp.bfloat16)
    gazeT = _prep(gaze_preds, jnp.bfloat16)
    dbT = jnp.pad(default_boxes.T, ((0, 0), (0, _AP - _A))).reshape(4, _S, _L)
    gtl = gt_labels.astype(jnp.int32).reshape(_B, 1, _G)

    out = pl.pallas_call(
        _body,
        grid=(_B,),
        in_specs=[
            pl.BlockSpec((1, _C, _S, _L), lambda b: (b, 0, 0, 0)),
            pl.BlockSpec((1, 4, _S, _L), lambda b: (b, 0, 0, 0)),
            pl.BlockSpec((1, 10, _S, _L), lambda b: (b, 0, 0, 0)),
            pl.BlockSpec((1, 2, _S, _L), lambda b: (b, 0, 0, 0)),
            pl.BlockSpec((4, _S, _L), lambda b: (0, 0, 0)),
            pl.BlockSpec((1, _G, 4), lambda b: (b, 0, 0),
                         memory_space=pltpu.SMEM),
            pl.BlockSpec((1, 1, _G), lambda b: (b, 0, 0),
                         memory_space=pltpu.SMEM),
            pl.BlockSpec((1, _G, 10), lambda b: (b, 0, 0),
                         memory_space=pltpu.SMEM),
            pl.BlockSpec((1, _G, 2), lambda b: (b, 0, 0),
                         memory_space=pltpu.SMEM),
        ],
        out_specs=pl.BlockSpec((5,), lambda b: (0,),
                               memory_space=pltpu.SMEM),
        out_shape=jax.ShapeDtypeStruct((5,), jnp.float32),
        scratch_shapes=[pltpu.SMEM((8,), jnp.float32),
                        pltpu.VMEM((_B, _S, _L), jnp.int32)],
    )(clsT, boxT, lmkT, gazeT, dbT, gt_boxes, gtl, gt_landmarks, gt_gaze)
    return out


# speedup vs baseline: 92.8797x; 1.0112x over previous
import jax
import jax.numpy as jnp
from jax.experimental import pallas as pl
from jax.experimental.pallas import tpu as pltpu

_B, _A, _G, _C = 16, 24564, 8, 3
_AP = 24576
_S, _L = 192, 128
_NEG_POS_RATIO = 3
_IOU_THR = 0.5
_I32_MIN = -2147483648
_I32_MAXP = 2147483647


def _smooth_l1(x):
    a = jnp.abs(x)
    return jnp.where(a < 1.0, 0.5 * a * a, a - 0.5)


def _body(cls_ref, blg_ref, db_ref, gtb, gtl, gtlmk, gtgz,
          out_ref, acc, s_all):
    b = pl.program_id(0)

    @pl.when(b == 0)
    def _init():
        for i in range(6):
            acc[i] = 0.0

    dcx = db_ref[0]
    dcy = db_ref[1]
    dw = db_ref[2]
    dh = db_ref[3]
    dx1 = dcx - dw / 2.0
    dy1 = dcy - dh / 2.0
    dx2 = dcx + dw / 2.0
    dy2 = dcy + dh / 2.0
    area_b = (dx2 - dx1) * (dy2 - dy1)

    sl = jax.lax.broadcasted_iota(jnp.int32, (_S, _L), 0)
    ll = jax.lax.broadcasted_iota(jnp.int32, (_S, _L), 1)
    aid = sl * _L + ll
    valid = aid < _A

    ious = []
    best_iou = None
    best_g = None
    for g in range(_G):
        gx1 = gtb[0, g, 0]
        gy1 = gtb[0, g, 1]
        gx2 = gtb[0, g, 2]
        gy2 = gtb[0, g, 3]
        w = jnp.maximum(jnp.minimum(gx2, dx2) - jnp.maximum(gx1, dx1), 0.0)
        h = jnp.maximum(jnp.minimum(gy2, dy2) - jnp.maximum(gy1, dy1), 0.0)
        inter = w * h
        area_a = (gx2 - gx1) * (gy2 - gy1)
        union = jnp.maximum(area_a + area_b - inter, 1e-8)
        iou = inter / union
        ious.append(iou)
        if g == 0:
            best_iou = iou
            best_g = jnp.zeros((_S, _L), jnp.int32)
        else:
            upd = iou > best_iou
            best_g = jnp.where(upd, g, best_g)
            best_iou = jnp.where(upd, iou, best_iou)

    am = jnp.full((_S, _L), -1, jnp.int32)
    for g in range(_G):
        mg = jnp.max(jnp.max(ious[g], axis=0))
        bai = jnp.min(jnp.min(jnp.where(ious[g] == mg, aid, _AP), axis=0))
        am = jnp.where(aid == bai, g, am)
    am = jnp.where(best_iou >= _IOU_THR, best_g, am)
    pos = am >= 0
    mi = jnp.where(pos, am, 0)

    clsv = jnp.zeros((_S, _L), jnp.int32)
    zf = jnp.zeros((_S, _L), jnp.float32)
    gcxv, gcyv, gwv, ghv = zf, zf, zf, zf
    lmkv = [zf] * 10
    gz0v, gz1v = zf, zf
    for g in range(_G):
        m = mi == g
        clsv = jnp.where(m, gtl[0, 0, g], clsv)
        gx1 = gtb[0, g, 0]
        gy1 = gtb[0, g, 1]
        gx2 = gtb[0, g, 2]
        gy2 = gtb[0, g, 3]
        gcxv = jnp.where(m, (gx1 + gx2) / 2.0, gcxv)
        gcyv = jnp.where(m, (gy1 + gy2) / 2.0, gcyv)
        gwv = jnp.where(m, gx2 - gx1, gwv)
        ghv = jnp.where(m, gy2 - gy1, ghv)
        for i in range(10):
            lmkv[i] = jnp.where(m, gtlmk[0, g, i], lmkv[i])
        gz0v = jnp.where(m, gtgz[0, g, 0], gz0v)
        gz1v = jnp.where(m, gtgz[0, g, 1], gz1v)

    classes = jnp.where(pos, clsv, 0)
    face = pos & (classes == 1)

    tx = (gcxv - dcx) / (dw * 0.1)
    ty = (gcyv - dcy) / (dh * 0.1)
    tw = jnp.log(jnp.maximum(gwv / dw, 1e-6)) / 0.2
    th = jnp.log(jnp.maximum(ghv / dh, 1e-6)) / 0.2
    bp = [blg_ref[0, j].astype(jnp.float32) for j in range(4)]
    box_v = (_smooth_l1(bp[0] - tx) + _smooth_l1(bp[1] - ty)
             + _smooth_l1(bp[2] - tw) + _smooth_l1(bp[3] - th))
    box_sum = jnp.sum(jnp.where(pos, box_v, 0.0))

    lmk_v = zf
    for i in range(5):
        ox = (lmkv[2 * i] - dcx) / (dw * 0.1)
        oy = (lmkv[2 * i + 1] - dcy) / (dh * 0.1)
        lmk_v = lmk_v + _smooth_l1(
            blg_ref[0, 4 + 2 * i].astype(jnp.float32) - ox)
        lmk_v = lmk_v + _smooth_l1(
            blg_ref[0, 5 + 2 * i].astype(jnp.float32) - oy)
    lmk_sum = jnp.sum(jnp.where(face, lmk_v, 0.0))

    gaze_v = (_smooth_l1(blg_ref[0, 14].astype(jnp.float32) - gz0v)
              + _smooth_l1(blg_ref[0, 15].astype(jnp.float32) - gz1v))
    gaze_sum = jnp.sum(jnp.where(face, gaze_v, 0.0))

    l0 = cls_ref[0, 0]
    l1 = cls_ref[0, 1]
    l2 = cls_ref[0, 2]
    mx = jnp.maximum(jnp.maximum(l0, l1), l2)
    lse = mx + jnp.log(jnp.exp(l0 - mx) + jnp.exp(l1 - mx) + jnp.exp(l2 - mx))
    lsel = jnp.where(classes == 1, l1, jnp.where(classes == 2, l2, l0))
    ce = lse - lsel

    nf_i = jnp.sum(jnp.where(face, 1, 0))
    cls_pos = jnp.sum(jnp.where(pos, ce, 0.0))

    si = jax.lax.bitcast_convert_type(ce, jnp.int32)
    s = jnp.where(si >= 0, si, si ^ _I32_MAXP)
    s = jnp.where(valid & (~pos), s, _I32_MIN)
    s_all[pl.ds(b, 1)] = s[None]

    acc[0] = acc[0] + cls_pos
    acc[1] = acc[1] + box_sum
    acc[2] = acc[2] + lmk_sum
    acc[3] = acc[3] + gaze_sum
    acc[5] = acc[5] + jnp.maximum(nf_i.astype(jnp.float32), 1.0)

    @pl.when(b == _B - 1)
    def _fin():
        sv = s_all[:]

        def _red(x):
            return jnp.sum(jnp.sum(x, axis=1), axis=1)

        n_min = _red(jnp.where(sv == _I32_MIN, 1, 0))
        np_vec = n_min - (_AP - _A)
        k_vec = jnp.minimum(_A - np_vec,
                            _NEG_POS_RATIO * jnp.maximum(np_vec, 1))
        p = jnp.full((_B,), _I32_MIN, jnp.int32)
        for bit in range(30, -1, -1):
            cand = p | jnp.int32(1 << bit)
            cand_s = (cand ^ _I32_MIN).reshape(_B, 1, 1)
            cnt = _red(jnp.where(sv >= cand_s, 1, 0))
            p = jnp.where(cnt >= k_vec, cand, p)
        ts = p ^ _I32_MIN
        ti = jnp.where(ts >= 0, ts, ts ^ _I32_MAXP)
        tf = jax.lax.bitcast_convert_type(ti, jnp.float32)
        tsb = ts.reshape(_B, 1, 1)
        gt_m = sv > tsb
        cnt_gt = _red(jnp.where(gt_m, 1, 0))
        cei = jnp.where(sv >= 0, sv, sv ^ _I32_MAXP)
        cef = jax.lax.bitcast_convert_type(cei, jnp.float32)
        sum_gt = jnp.sum(jnp.sum(jnp.where(gt_m, cef, 0.0), axis=1), axis=1)
        neg_vec = jnp.where(k_vec > 0,
                            sum_gt + (k_vec - cnt_gt).astype(jnp.float32) * tf,
                            0.0)
        neg_total = jnp.sum(neg_vec)
        npf_total = jnp.sum(jnp.maximum(np_vec, 1).astype(jnp.float32))

        tcls = (acc[0] + neg_total) / npf_total
        tbox = acc[1] / npf_total
        tlmk = acc[2] / acc[5]
        tgaze = acc[3] / acc[5]
        out_ref[0] = tcls + tbox + 0.5 * tlmk + 0.5 * tgaze
        out_ref[1] = tcls
        out_ref[2] = tbox
        out_ref[3] = tlmk
        out_ref[4] = tgaze


def _prep(x, dtype=jnp.float32):
    k = x.shape[-1]
    xt = jnp.transpose(x, (0, 2, 1)).astype(dtype)
    xt = jnp.pad(xt, ((0, 0), (0, 0), (0, _AP - _A)))
    return xt.reshape(_B, k, _S, _L)


def kernel(cls_logits, box_preds, lmk_preds, gaze_preds, default_boxes,
           gt_boxes, gt_landmarks, gt_gaze, gt_labels):
    clsT = _prep(cls_logits)
    blgT = _prep(jnp.concatenate([box_preds, lmk_preds, gaze_preds], axis=2),
                 jnp.bfloat16)
    dbT = jnp.pad(default_boxes.T, ((0, 0), (0, _AP - _A))).reshape(4, _S, _L)
    gtl = gt_labels.astype(jnp.int32).reshape(_B, 1, _G)

    out = pl.pallas_call(
        _body,
        grid=(_B,),
        in_specs=[
            pl.BlockSpec((1, _C, _S, _L), lambda b: (b, 0, 0, 0)),
            pl.BlockSpec((1, 16, _S, _L), lambda b: (b, 0, 0, 0)),
            pl.BlockSpec((4, _S, _L), lambda b: (0, 0, 0)),
            pl.BlockSpec((1, _G, 4), lambda b: (b, 0, 0),
                         memory_space=pltpu.SMEM),
            pl.BlockSpec((1, 1, _G), lambda b: (b, 0, 0),
                         memory_space=pltpu.SMEM),
            pl.BlockSpec((1, _G, 10), lambda b: (b, 0, 0),
                         memory_space=pltpu.SMEM),
            pl.BlockSpec((1, _G, 2), lambda b: (b, 0, 0),
                         memory_space=pltpu.SMEM),
        ],
        out_specs=pl.BlockSpec((5,), lambda b: (0,),
                               memory_space=pltpu.SMEM),
        out_shape=jax.ShapeDtypeStruct((5,), jnp.float32),
        scratch_shapes=[pltpu.SMEM((8,), jnp.float32),
                        pltpu.VMEM((_B, _S, _L), jnp.int32)],
    )(clsT, blgT, dbT, gt_boxes, gtl, gt_landmarks, gt_gaze)
    return out


# speedup vs baseline: 93.0725x; 1.0021x over previous
import jax
import jax.numpy as jnp
from jax.experimental import pallas as pl
from jax.experimental.pallas import tpu as pltpu

_B, _A, _G, _C = 16, 24564, 8, 3
_AP = 24576
_S, _L = 192, 128
_NEG_POS_RATIO = 3
_IOU_THR = 0.5
_I32_MIN = -2147483648
_I32_MAXP = 2147483647


def _smooth_l1(x):
    a = jnp.abs(x)
    return jnp.where(a < 1.0, 0.5 * a * a, a - 0.5)


def _body(cls_ref, blg_ref, db_ref, gtb, gtl, gtlmk, gtgz,
          out_ref, acc, s_all):
    b = pl.program_id(0)

    @pl.when(b == 0)
    def _init():
        for i in range(6):
            acc[i] = 0.0

    dcx = db_ref[0]
    dcy = db_ref[1]
    dw = db_ref[2]
    dh = db_ref[3]
    dx1 = dcx - dw / 2.0
    dy1 = dcy - dh / 2.0
    dx2 = dcx + dw / 2.0
    dy2 = dcy + dh / 2.0
    area_b = (dx2 - dx1) * (dy2 - dy1)

    sl = jax.lax.broadcasted_iota(jnp.int32, (_S, _L), 0)
    ll = jax.lax.broadcasted_iota(jnp.int32, (_S, _L), 1)
    aid = sl * _L + ll
    valid = aid < _A

    ious = []
    best_iou = None
    best_g = None
    for g in range(_G):
        gx1 = gtb[0, g, 0]
        gy1 = gtb[0, g, 1]
        gx2 = gtb[0, g, 2]
        gy2 = gtb[0, g, 3]
        w = jnp.maximum(jnp.minimum(gx2, dx2) - jnp.maximum(gx1, dx1), 0.0)
        h = jnp.maximum(jnp.minimum(gy2, dy2) - jnp.maximum(gy1, dy1), 0.0)
        inter = w * h
        area_a = (gx2 - gx1) * (gy2 - gy1)
        union = jnp.maximum(area_a + area_b - inter, 1e-8)
        iou = inter / union
        ious.append(iou)
        if g == 0:
            best_iou = iou
            best_g = jnp.zeros((_S, _L), jnp.int32)
        else:
            upd = iou > best_iou
            best_g = jnp.where(upd, g, best_g)
            best_iou = jnp.where(upd, iou, best_iou)

    am = jnp.full((_S, _L), -1, jnp.int32)
    for g in range(_G):
        mg = jnp.max(jnp.max(ious[g], axis=0))
        bai = jnp.min(jnp.min(jnp.where(ious[g] == mg, aid, _AP), axis=0))
        am = jnp.where(aid == bai, g, am)
    am = jnp.where(best_iou >= _IOU_THR, best_g, am)
    pos = am >= 0
    mi = jnp.where(pos, am, 0)

    clsv = jnp.zeros((_S, _L), jnp.int32)
    zf = jnp.zeros((_S, _L), jnp.float32)
    gcxv, gcyv, gwv, ghv = zf, zf, zf, zf
    lmkv = [zf] * 10
    gz0v, gz1v = zf, zf
    for g in range(_G):
        m = mi == g
        clsv = jnp.where(m, gtl[0, 0, g], clsv)
        gx1 = gtb[0, g, 0]
        gy1 = gtb[0, g, 1]
        gx2 = gtb[0, g, 2]
        gy2 = gtb[0, g, 3]
        gcxv = jnp.where(m, (gx1 + gx2) / 2.0, gcxv)
        gcyv = jnp.where(m, (gy1 + gy2) / 2.0, gcyv)
        gwv = jnp.where(m, gx2 - gx1, gwv)
        ghv = jnp.where(m, gy2 - gy1, ghv)
        for i in range(10):
            lmkv[i] = jnp.where(m, gtlmk[0, g, i], lmkv[i])
        gz0v = jnp.where(m, gtgz[0, g, 0], gz0v)
        gz1v = jnp.where(m, gtgz[0, g, 1], gz1v)

    classes = jnp.where(pos, clsv, 0)
    face = pos & (classes == 1)

    tx = (gcxv - dcx) / (dw * 0.1)
    ty = (gcyv - dcy) / (dh * 0.1)
    tw = jnp.log(jnp.maximum(gwv / dw, 1e-6)) / 0.2
    th = jnp.log(jnp.maximum(ghv / dh, 1e-6)) / 0.2
    bp = [blg_ref[0, j].astype(jnp.float32) for j in range(4)]
    box_v = (_smooth_l1(bp[0] - tx) + _smooth_l1(bp[1] - ty)
             + _smooth_l1(bp[2] - tw) + _smooth_l1(bp[3] - th))
    box_sum = jnp.sum(jnp.where(pos, box_v, 0.0))

    lmk_v = zf
    for i in range(5):
        ox = (lmkv[2 * i] - dcx) / (dw * 0.1)
        oy = (lmkv[2 * i + 1] - dcy) / (dh * 0.1)
        lmk_v = lmk_v + _smooth_l1(
            blg_ref[0, 4 + 2 * i].astype(jnp.float32) - ox)
        lmk_v = lmk_v + _smooth_l1(
            blg_ref[0, 5 + 2 * i].astype(jnp.float32) - oy)
    lmk_sum = jnp.sum(jnp.where(face, lmk_v, 0.0))

    gaze_v = (_smooth_l1(blg_ref[0, 14].astype(jnp.float32) - gz0v)
              + _smooth_l1(blg_ref[0, 15].astype(jnp.float32) - gz1v))
    gaze_sum = jnp.sum(jnp.where(face, gaze_v, 0.0))

    l0 = cls_ref[0, 0].astype(jnp.float32)
    l1 = cls_ref[0, 1].astype(jnp.float32)
    l2 = cls_ref[0, 2].astype(jnp.float32)
    mx = jnp.maximum(jnp.maximum(l0, l1), l2)
    lse = mx + jnp.log(jnp.exp(l0 - mx) + jnp.exp(l1 - mx) + jnp.exp(l2 - mx))
    lsel = jnp.where(classes == 1, l1, jnp.where(classes == 2, l2, l0))
    ce = lse - lsel

    nf_i = jnp.sum(jnp.where(face, 1, 0))
    cls_pos = jnp.sum(jnp.where(pos, ce, 0.0))

    si = jax.lax.bitcast_convert_type(ce, jnp.int32)
    s = jnp.where(si >= 0, si, si ^ _I32_MAXP)
    s = jnp.where(valid & (~pos), s, _I32_MIN)
    s_all[pl.ds(b, 1)] = s[None]

    acc[0] = acc[0] + cls_pos
    acc[1] = acc[1] + box_sum
    acc[2] = acc[2] + lmk_sum
    acc[3] = acc[3] + gaze_sum
    acc[5] = acc[5] + jnp.maximum(nf_i.astype(jnp.float32), 1.0)

    @pl.when(b == _B - 1)
    def _fin():
        sv = s_all[:]

        def _red(x):
            return jnp.sum(jnp.sum(x, axis=1), axis=1)

        n_min = _red(jnp.where(sv == _I32_MIN, 1, 0))
        np_vec = n_min - (_AP - _A)
        k_vec = jnp.minimum(_A - np_vec,
                            _NEG_POS_RATIO * jnp.maximum(np_vec, 1))
        p = jnp.full((_B,), _I32_MIN, jnp.int32)
        for bit in range(30, -1, -1):
            cand = p | jnp.int32(1 << bit)
            cand_s = (cand ^ _I32_MIN).reshape(_B, 1, 1)
            cnt = _red(jnp.where(sv >= cand_s, 1, 0))
            p = jnp.where(cnt >= k_vec, cand, p)
        ts = p ^ _I32_MIN
        ti = jnp.where(ts >= 0, ts, ts ^ _I32_MAXP)
        tf = jax.lax.bitcast_convert_type(ti, jnp.float32)
        tsb = ts.reshape(_B, 1, 1)
        gt_m = sv > tsb
        cnt_gt = _red(jnp.where(gt_m, 1, 0))
        cei = jnp.where(sv >= 0, sv, sv ^ _I32_MAXP)
        cef = jax.lax.bitcast_convert_type(cei, jnp.float32)
        sum_gt = jnp.sum(jnp.sum(jnp.where(gt_m, cef, 0.0), axis=1), axis=1)
        neg_vec = jnp.where(k_vec > 0,
                            sum_gt + (k_vec - cnt_gt).astype(jnp.float32) * tf,
                            0.0)
        neg_total = jnp.sum(neg_vec)
        npf_total = jnp.sum(jnp.maximum(np_vec, 1).astype(jnp.float32))

        tcls = (acc[0] + neg_total) / npf_total
        tbox = acc[1] / npf_total
        tlmk = acc[2] / acc[5]
        tgaze = acc[3] / acc[5]
        out_ref[0] = tcls + tbox + 0.5 * tlmk + 0.5 * tgaze
        out_ref[1] = tcls
        out_ref[2] = tbox
        out_ref[3] = tlmk
        out_ref[4] = tgaze


def _prep(x, dtype=jnp.float32):
    k = x.shape[-1]
    xt = jnp.transpose(x, (0, 2, 1)).astype(dtype)
    xt = jnp.pad(xt, ((0, 0), (0, 0), (0, _AP - _A)))
    return xt.reshape(_B, k, _S, _L)


def kernel(cls_logits, box_preds, lmk_preds, gaze_preds, default_boxes,
           gt_boxes, gt_landmarks, gt_gaze, gt_labels):
    clsT = _prep(cls_logits, jnp.bfloat16)
    blgT = _prep(jnp.concatenate([box_preds, lmk_preds, gaze_preds], axis=2),
                 jnp.bfloat16)
    dbT = jnp.pad(default_boxes.T, ((0, 0), (0, _AP - _A))).reshape(4, _S, _L)
    gtl = gt_labels.astype(jnp.int32).reshape(_B, 1, _G)

    out = pl.pallas_call(
        _body,
        grid=(_B,),
        in_specs=[
            pl.BlockSpec((1, _C, _S, _L), lambda b: (b, 0, 0, 0)),
            pl.BlockSpec((1, 16, _S, _L), lambda b: (b, 0, 0, 0)),
            pl.BlockSpec((4, _S, _L), lambda b: (0, 0, 0)),
            pl.BlockSpec((1, _G, 4), lambda b: (b, 0, 0),
                         memory_space=pltpu.SMEM),
            pl.BlockSpec((1, 1, _G), lambda b: (b, 0, 0),
                         memory_space=pltpu.SMEM),
            pl.BlockSpec((1, _G, 10), lambda b: (b, 0, 0),
                         memory_space=pltpu.SMEM),
            pl.BlockSpec((1, _G, 2), lambda b: (b, 0, 0),
                         memory_space=pltpu.SMEM),
        ],
        out_specs=pl.BlockSpec((5,), lambda b: (0,),
                               memory_space=pltpu.SMEM),
        out_shape=jax.ShapeDtypeStruct((5,), jnp.float32),
        scratch_shapes=[pltpu.SMEM((8,), jnp.float32),
                        pltpu.VMEM((_B, _S, _L), jnp.int32)],
    )(clsT, blgT, dbT, gt_boxes, gtl, gt_landmarks, gt_gaze)
    return out


# speedup vs baseline: 93.1579x; 1.0009x over previous
import jax
import jax.numpy as jnp
from jax.experimental import pallas as pl
from jax.experimental.pallas import tpu as pltpu

_B, _A, _G, _C = 16, 24564, 8, 3
_AP = 24576
_S, _L = 192, 128
_NEG_POS_RATIO = 3
_IOU_THR = 0.5
_I32_MIN = -2147483648
_I32_MAXP = 2147483647


def _smooth_l1(x):
    a = jnp.abs(x)
    return jnp.where(a < 1.0, 0.5 * a * a, a - 0.5)


def _body(cls_ref, blg_ref, db_ref, gtb, gtl, gtlmk, gtgz,
          out_ref, acc, s_all):
    b = pl.program_id(0)

    @pl.when(b == 0)
    def _init():
        for i in range(6):
            acc[i] = 0.0

    dcx = db_ref[0]
    dcy = db_ref[1]
    dw = db_ref[2]
    dh = db_ref[3]
    dx1 = dcx - dw / 2.0
    dy1 = dcy - dh / 2.0
    dx2 = dcx + dw / 2.0
    dy2 = dcy + dh / 2.0
    area_b = (dx2 - dx1) * (dy2 - dy1)

    sl = jax.lax.broadcasted_iota(jnp.int32, (_S, _L), 0)
    ll = jax.lax.broadcasted_iota(jnp.int32, (_S, _L), 1)
    aid = sl * _L + ll
    valid = aid < _A

    ious = []
    best_iou = None
    best_g = None
    for g in range(_G):
        gx1 = gtb[0, g, 0]
        gy1 = gtb[0, g, 1]
        gx2 = gtb[0, g, 2]
        gy2 = gtb[0, g, 3]
        w = jnp.maximum(jnp.minimum(gx2, dx2) - jnp.maximum(gx1, dx1), 0.0)
        h = jnp.maximum(jnp.minimum(gy2, dy2) - jnp.maximum(gy1, dy1), 0.0)
        inter = w * h
        area_a = (gx2 - gx1) * (gy2 - gy1)
        union = jnp.maximum(area_a + area_b - inter, 1e-8)
        iou = inter / union
        ious.append(iou)
        if g == 0:
            best_iou = iou
            best_g = jnp.zeros((_S, _L), jnp.int32)
        else:
            upd = iou > best_iou
            best_g = jnp.where(upd, g, best_g)
            best_iou = jnp.where(upd, iou, best_iou)

    am = jnp.full((_S, _L), -1, jnp.int32)
    for g in range(_G):
        mg = jnp.max(jnp.max(ious[g], axis=0))
        bai = jnp.min(jnp.min(jnp.where(ious[g] == mg, aid, _AP), axis=0))
        am = jnp.where(aid == bai, g, am)
    am = jnp.where(best_iou >= _IOU_THR, best_g, am)
    pos = am >= 0
    mi = jnp.where(pos, am, 0)

    clsv = jnp.zeros((_S, _L), jnp.int32)
    zf = jnp.zeros((_S, _L), jnp.float32)
    gcxv, gcyv, gwv, ghv = zf, zf, zf, zf
    lmkv = [zf] * 10
    gz0v, gz1v = zf, zf
    for g in range(_G):
        m = mi == g
        clsv = jnp.where(m, gtl[0, 0, g], clsv)
        gx1 = gtb[0, g, 0]
        gy1 = gtb[0, g, 1]
        gx2 = gtb[0, g, 2]
        gy2 = gtb[0, g, 3]
        gcxv = jnp.where(m, (gx1 + gx2) / 2.0, gcxv)
        gcyv = jnp.where(m, (gy1 + gy2) / 2.0, gcyv)
        gwv = jnp.where(m, gx2 - gx1, gwv)
        ghv = jnp.where(m, gy2 - gy1, ghv)
        for i in range(10):
            lmkv[i] = jnp.where(m, gtlmk[0, g, i], lmkv[i])
        gz0v = jnp.where(m, gtgz[0, g, 0], gz0v)
        gz1v = jnp.where(m, gtgz[0, g, 1], gz1v)

    classes = jnp.where(pos, clsv, 0)
    face = pos & (classes == 1)

    rw = 1.0 / (dw * 0.1)
    rh = 1.0 / (dh * 0.1)
    tx = (gcxv - dcx) * rw
    ty = (gcyv - dcy) * rh
    tw = jnp.log(jnp.maximum(gwv * (rw * 0.1), 1e-6)) * 5.0
    th = jnp.log(jnp.maximum(ghv * (rh * 0.1), 1e-6)) * 5.0
    bp = [blg_ref[0, j].astype(jnp.float32) for j in range(4)]
    box_v = (_smooth_l1(bp[0] - tx) + _smooth_l1(bp[1] - ty)
             + _smooth_l1(bp[2] - tw) + _smooth_l1(bp[3] - th))
    box_sum = jnp.sum(jnp.where(pos, box_v, 0.0))

    lmk_v = zf
    for i in range(5):
        ox = (lmkv[2 * i] - dcx) * rw
        oy = (lmkv[2 * i + 1] - dcy) * rh
        lmk_v = lmk_v + _smooth_l1(
            blg_ref[0, 4 + 2 * i].astype(jnp.float32) - ox)
        lmk_v = lmk_v + _smooth_l1(
            blg_ref[0, 5 + 2 * i].astype(jnp.float32) - oy)
    lmk_sum = jnp.sum(jnp.where(face, lmk_v, 0.0))

    gaze_v = (_smooth_l1(blg_ref[0, 14].astype(jnp.float32) - gz0v)
              + _smooth_l1(blg_ref[0, 15].astype(jnp.float32) - gz1v))
    gaze_sum = jnp.sum(jnp.where(face, gaze_v, 0.0))

    l0 = cls_ref[0, 0].astype(jnp.float32)
    l1 = cls_ref[0, 1].astype(jnp.float32)
    l2 = cls_ref[0, 2].astype(jnp.float32)
    mx = jnp.maximum(jnp.maximum(l0, l1), l2)
    lse = mx + jnp.log(jnp.exp(l0 - mx) + jnp.exp(l1 - mx) + jnp.exp(l2 - mx))
    lsel = jnp.where(classes == 1, l1, jnp.where(classes == 2, l2, l0))
    ce = lse - lsel

    nf_i = jnp.sum(jnp.where(face, 1, 0))
    cls_pos = jnp.sum(jnp.where(pos, ce, 0.0))

    si = jax.lax.bitcast_convert_type(ce, jnp.int32)
    s = jnp.where(si >= 0, si, si ^ _I32_MAXP)
    s = jnp.where(valid & (~pos), s, _I32_MIN)
    s_all[pl.ds(b, 1)] = s[None]

    acc[0] = acc[0] + cls_pos
    acc[1] = acc[1] + box_sum
    acc[2] = acc[2] + lmk_sum
    acc[3] = acc[3] + gaze_sum
    acc[5] = acc[5] + jnp.maximum(nf_i.astype(jnp.float32), 1.0)

    @pl.when(b == _B - 1)
    def _fin():
        sv = s_all[:]

        def _red(x):
            return jnp.sum(jnp.sum(x, axis=1), axis=1)

        n_min = _red(jnp.where(sv == _I32_MIN, 1, 0))
        np_vec = n_min - (_AP - _A)
        k_vec = jnp.minimum(_A - np_vec,
                            _NEG_POS_RATIO * jnp.maximum(np_vec, 1))
        p = jnp.full((_B,), _I32_MIN, jnp.int32)
        for bit in range(30, -1, -1):
            cand = p | jnp.int32(1 << bit)
            cand_s = (cand ^ _I32_MIN).reshape(_B, 1, 1)
            cnt = _red(jnp.where(sv >= cand_s, 1, 0))
            p = jnp.where(cnt >= k_vec, cand, p)
        ts = p ^ _I32_MIN
        ti = jnp.where(ts >= 0, ts, ts ^ _I32_MAXP)
        tf = jax.lax.bitcast_convert_type(ti, jnp.float32)
        tsb = ts.reshape(_B, 1, 1)
        gt_m = sv > tsb
        cnt_gt = _red(jnp.where(gt_m, 1, 0))
        cei = jnp.where(sv >= 0, sv, sv ^ _I32_MAXP)
        cef = jax.lax.bitcast_convert_type(cei, jnp.float32)
        sum_gt = jnp.sum(jnp.sum(jnp.where(gt_m, cef, 0.0), axis=1), axis=1)
        neg_vec = jnp.where(k_vec > 0,
                            sum_gt + (k_vec - cnt_gt).astype(jnp.float32) * tf,
                            0.0)
        neg_total = jnp.sum(neg_vec)
        npf_total = jnp.sum(jnp.maximum(np_vec, 1).astype(jnp.float32))

        tcls = (acc[0] + neg_total) / npf_total
        tbox = acc[1] / npf_total
        tlmk = acc[2] / acc[5]
        tgaze = acc[3] / acc[5]
        out_ref[0] = tcls + tbox + 0.5 * tlmk + 0.5 * tgaze
        out_ref[1] = tcls
        out_ref[2] = tbox
        out_ref[3] = tlmk
        out_ref[4] = tgaze


def _prep(x, dtype=jnp.float32):
    k = x.shape[-1]
    xt = jnp.transpose(x, (0, 2, 1)).astype(dtype)
    xt = jnp.pad(xt, ((0, 0), (0, 0), (0, _AP - _A)))
    return xt.reshape(_B, k, _S, _L)


def kernel(cls_logits, box_preds, lmk_preds, gaze_preds, default_boxes,
           gt_boxes, gt_landmarks, gt_gaze, gt_labels):
    clsT = _prep(cls_logits, jnp.bfloat16)
    blgT = _prep(jnp.concatenate([box_preds, lmk_preds, gaze_preds], axis=2),
                 jnp.bfloat16)
    dbT = jnp.pad(default_boxes.T, ((0, 0), (0, _AP - _A))).reshape(4, _S, _L)
    gtl = gt_labels.astype(jnp.int32).reshape(_B, 1, _G)

    out = pl.pallas_call(
        _body,
        grid=(_B,),
        in_specs=[
            pl.BlockSpec((1, _C, _S, _L), lambda b: (b, 0, 0, 0)),
            pl.BlockSpec((1, 16, _S, _L), lambda b: (b, 0, 0, 0)),
            pl.BlockSpec((4, _S, _L), lambda b: (0, 0, 0)),
            pl.BlockSpec((1, _G, 4), lambda b: (b, 0, 0),
                         memory_space=pltpu.SMEM),
            pl.BlockSpec((1, 1, _G), lambda b: (b, 0, 0),
                         memory_space=pltpu.SMEM),
            pl.BlockSpec((1, _G, 10), lambda b: (b, 0, 0),
                         memory_space=pltpu.SMEM),
            pl.BlockSpec((1, _G, 2), lambda b: (b, 0, 0),
                         memory_space=pltpu.SMEM),
        ],
        out_specs=pl.BlockSpec((5,), lambda b: (0,),
                               memory_space=pltpu.SMEM),
        out_shape=jax.ShapeDtypeStruct((5,), jnp.float32),
        scratch_shapes=[pltpu.SMEM((8,), jnp.float32),
                        pltpu.VMEM((_B, _S, _L), jnp.int32)],
    )(clsT, blgT, dbT, gt_boxes, gtl, gt_landmarks, gt_gaze)
    return out
